# Initial kernel scaffold; baseline (speedup 1.0000x reference)
import jax
import jax.numpy as jnp
from jax.experimental import pallas as pl


def kernel(x, pos, W0, b0, W1, b1, W2, b2, Wf, bf):
    raise NotImplementedError("write your pallas kernel here")


# speedup vs baseline: 1.3416x; 1.3416x over previous
import functools

import jax
import jax.numpy as jnp
from jax.experimental import pallas as pl

N = 10000
K = 20
H = 128


def _layer_body(h_ref, mx_ref, pp_ref, wx_ref, wd_ref, wp_ref, b_ref, o_ref):
    h = h_ref[...]
    mx = mx_ref[...]
    acc = jnp.dot(mx, wx_ref[...], preferred_element_type=jnp.float32,
                  precision=jax.lax.Precision.HIGHEST)
    acc += jnp.dot(h - mx, wd_ref[...], preferred_element_type=jnp.float32,
                   precision=jax.lax.Precision.HIGHEST)
    acc += jnp.dot(pp_ref[...], wp_ref[...], preferred_element_type=jnp.float32,
                   precision=jax.lax.Precision.HIGHEST)
    o_ref[...] = jnp.maximum(acc + b_ref[...], 0.0)


@functools.partial(jax.jit, static_argnames=("blk",))
def _layer(h, mx, pp, wx, wd, wp, b, blk=1000):
    n = h.shape[0]
    grid = (n // blk,)
    return pl.pallas_call(
        _layer_body,
        grid=grid,
        in_specs=[
            pl.BlockSpec((blk, H), lambda i: (i, 0)),
            pl.BlockSpec((blk, H), lambda i: (i, 0)),
            pl.BlockSpec((blk, 8), lambda i: (i, 0)),
            pl.BlockSpec((H, H), lambda i: (0, 0)),
            pl.BlockSpec((H, H), lambda i: (0, 0)),
            pl.BlockSpec((8, H), lambda i: (0, 0)),
            pl.BlockSpec((1, H), lambda i: (0, 0)),
        ],
        out_specs=pl.BlockSpec((blk, H), lambda i: (i, 0)),
        out_shape=jax.ShapeDtypeStruct((n, H), jnp.float32),
    )(h, mx, pp, wx, wd, wp, b)


def _final_body(h_ref, wf_ref, o_ref):
    h = h_ref[...]
    wf = wf_ref[...]
    logits = jnp.sum(h * wf, axis=1, keepdims=True)
    m = jnp.max(logits)
    e = jnp.exp(logits - m)
    o_ref[...] = e / jnp.sum(e)


def _final(h, wf):
    n = h.shape[0]
    return pl.pallas_call(
        _final_body,
        out_shape=jax.ShapeDtypeStruct((n, 1), jnp.float32),
    )(h, wf.reshape(1, H))


def kernel(x, pos, W0, b0, W1, b1, W2, b2, Wf, bf):
    sq = jnp.sum(pos * pos, axis=1)
    dist = sq[:, None] + sq[None, :] - 2.0 * (pos @ pos.T)
    dist = dist.at[jnp.arange(N), jnp.arange(N)].set(jnp.inf)
    _, idx = jax.lax.top_k(-dist, K)

    nbr2 = idx[idx]
    mut = jnp.any(nbr2 == jnp.arange(N)[:, None, None], axis=-1)
    w = 1.0 - mut.astype(jnp.float32)
    flat_t = idx.reshape(-1)
    cnt = K + jax.ops.segment_sum(w.reshape(-1), flat_t, num_segments=N)

    def agg(h):
        g = h[idx].sum(1)
        s = jax.ops.segment_sum(
            jnp.repeat(h, K, axis=0) * w.reshape(-1)[:, None],
            flat_t, num_segments=N)
        return (g + s) / cnt[:, None]

    m_p = agg(pos)
    pp = jnp.pad(pos - m_p, ((0, 0), (0, 5)))

    h = x
    for W, b in [(W0, b0), (W1, b1), (W2, b2)]:
        d = h.shape[1]
        wx, wd = W[:d], W[d:2 * d]
        wp = jnp.pad(W[2 * d:], ((0, 5), (0, 0)))
        m_x = agg(h)
        h = _layer(h, m_x, pp, wx, wd, wp, b.reshape(1, H))

    return _final(h, Wf).reshape(N)


# speedup vs baseline: 2.2607x; 1.6851x over previous
import functools

import jax
import jax.numpy as jnp
from jax import lax
from jax.experimental import pallas as pl
from jax.experimental.pallas import tpu as pltpu
from jax.experimental.pallas import tpu_sc as plsc

N = 10000
K = 20
H = 128

NP = 10016
RPW = NP // 32
OW = 24
NV = N // 16
CAND = 512
BIG = 3.0e38


def _knn_body(px_h, py_h, pz_h, sq_h, out_h, px, py, pz, sq, drow,
              cd, ci, ob):
    wid = lax.axis_index("s") * 2 + lax.axis_index("c")
    pltpu.sync_copy(px_h, px)
    pltpu.sync_copy(py_h, py)
    pltpu.sync_copy(pz_h, pz)
    pltpu.sync_copy(sq_h, sq)

    iota = lax.iota(jnp.int32, 16)
    infv = jnp.full((16,), BIG, jnp.float32)
    base = wid * RPW

    def shuf(v, lanes):
        return lax.gather(
            v, lanes[:, None],
            lax.GatherDimensionNumbers(offset_dims=(), collapsed_slice_dims=(0,),
                                       start_index_map=(0,)),
            (1,), mode=lax.GatherScatterMode.PROMISE_IN_BOUNDS)

    def splat_at(ref, p):
        blk = (p // 16) * 16
        v = ref[pl.ds(blk, 16)]
        return shuf(v, jnp.full((16,), p - blk, jnp.int32))

    def vmin_splat(v):
        for sh in (8, 4, 2, 1):
            v = jnp.minimum(v, shuf(v, iota ^ sh))
        return v

    def prefix_sum(x):
        for sh in (1, 2, 4, 8):
            x = x + jnp.where(iota >= sh, shuf(x, jnp.maximum(iota - sh, 0)), 0)
        return x

    def row_body(rl, _):
        r = base + rl
        s = splat_at(sq, r)
        a = -2.0 * splat_at(px, r)
        b = -2.0 * splat_at(py, r)
        c = -2.0 * splat_at(pz, r)

        def dpair(j):
            jb = 16 * j
            xv = px[pl.ds(jb, 16)]
            yv = py[pl.ds(jb, 16)]
            zv = pz[pl.ds(jb, 16)]
            tv = sq[pl.ds(jb, 16)]
            d = (tv + s) + ((a * xv + b * yv) + c * zv)
            jv = iota + jb
            d = jnp.where(jv == r, BIG, d)
            drow[pl.ds(jb, 16)] = d
            return d

        def dbody(i, carry):
            m1, m2 = carry
            m1 = jnp.minimum(m1, dpair(2 * i))
            m2 = jnp.minimum(m2, dpair(2 * i + 1))
            return m1, m2

        m1, m2 = lax.fori_loop(0, NV // 2, dbody, (infv, infv))
        m1 = jnp.minimum(m1, dpair(NV - 1))

        tau = infv
        for _k in range(K):
            t1 = vmin_splat(m1)
            t2 = vmin_splat(m2)
            use1 = t1 <= t2
            t = jnp.minimum(t1, t2)
            tgt = jnp.where(use1, m1, m2)
            lane = vmin_splat(jnp.where(tgt == t, iota, 16))
            nt = jnp.where(iota == lane, BIG, tgt)
            m1 = jnp.where(use1, nt, m1)
            m2 = jnp.where(use1, m2, nt)
            tau = t

        def pbody(v, _):
            cd[pl.ds(16 * v, 16)] = infv
            return 0

        lax.fori_loop(0, CAND // 16, pbody, 0)

        zerov = jnp.zeros((16,), jnp.int32)
        lane15 = jnp.full((16,), 15, jnp.int32)

        def cbody(v, off):
            jb = 16 * v
            d = drow[pl.ds(jb, 16)]
            mask = d <= tau
            csum = prefix_sum(jnp.where(mask, 1, 0))
            pos = jnp.minimum(off + csum - 1, CAND - 1)
            plsc.store_scatter(cd, [pos], d, mask=mask)
            plsc.store_scatter(ci, [pos], iota + jb, mask=mask)
            return off + shuf(csum, lane15)

        nc = lax.fori_loop(0, NV, cbody, zerov)
        nv = ((jnp.minimum(nc, CAND) + 15) // 16)[0]

        def fold(v, m):
            return jnp.minimum(m, cd[pl.ds(16 * v, 16)])

        lane0 = iota == 0
        for k in range(K):
            m = lax.fori_loop(0, nv, fold, infv)
            t = vmin_splat(m)

            def sbody(v, best):
                mask = cd[pl.ds(16 * v, 16)] == t
                lane = vmin_splat(jnp.where(mask, iota, 16))
                p = 16 * v + lane
                return jnp.minimum(best, jnp.where(lane < 16, p, 1 << 30))

            pv = lax.fori_loop(0, nv, sbody, jnp.full((16,), 1 << 30, jnp.int32))
            idxs = splat_at(ci, pv[0])
            plsc.store_scatter(ob, [jnp.full((16,), rl * OW + k, jnp.int32)],
                               idxs, mask=lane0)
            plsc.store_scatter(cd, [pv], infv, mask=lane0)
        return 0

    lax.fori_loop(0, RPW, row_body, 0)
    pltpu.sync_copy(ob, out_h.at[pl.ds(base * OW, RPW * OW)])


@jax.jit
def _knn_sc(px, py, pz, sq):
    mesh = plsc.VectorSubcoreMesh(core_axis_name="c", subcore_axis_name="s")
    fn = pl.kernel(
        _knn_body,
        mesh=mesh,
        compiler_params=pltpu.CompilerParams(needs_layout_passes=False),
        out_type=jax.ShapeDtypeStruct((NP * OW,), jnp.int32),
        scratch_types=[
            pltpu.VMEM((NP,), jnp.float32),
            pltpu.VMEM((NP,), jnp.float32),
            pltpu.VMEM((NP,), jnp.float32),
            pltpu.VMEM((NP,), jnp.float32),
            pltpu.VMEM((N,), jnp.float32),
            pltpu.VMEM((CAND,), jnp.float32),
            pltpu.VMEM((CAND,), jnp.int32),
            pltpu.VMEM((RPW * OW,), jnp.int32),
        ],
    )
    return fn(px, py, pz, sq)


def _layer_body(h_ref, mx_ref, pp_ref, wx_ref, wd_ref, wp_ref, b_ref, o_ref):
    h = h_ref[...]
    mx = mx_ref[...]
    acc = jnp.dot(mx, wx_ref[...], preferred_element_type=jnp.float32,
                  precision=lax.Precision.HIGHEST)
    acc += jnp.dot(h - mx, wd_ref[...], preferred_element_type=jnp.float32,
                   precision=lax.Precision.HIGHEST)
    acc += jnp.dot(pp_ref[...], wp_ref[...], preferred_element_type=jnp.float32,
                   precision=lax.Precision.HIGHEST)
    o_ref[...] = jnp.maximum(acc + b_ref[...], 0.0)


@functools.partial(jax.jit, static_argnames=("blk",))
def _layer(h, mx, pp, wx, wd, wp, b, blk=1000):
    n = h.shape[0]
    grid = (n // blk,)
    return pl.pallas_call(
        _layer_body,
        grid=grid,
        in_specs=[
            pl.BlockSpec((blk, H), lambda i: (i, 0)),
            pl.BlockSpec((blk, H), lambda i: (i, 0)),
            pl.BlockSpec((blk, 8), lambda i: (i, 0)),
            pl.BlockSpec((H, H), lambda i: (0, 0)),
            pl.BlockSpec((H, H), lambda i: (0, 0)),
            pl.BlockSpec((8, H), lambda i: (0, 0)),
            pl.BlockSpec((1, H), lambda i: (0, 0)),
        ],
        out_specs=pl.BlockSpec((blk, H), lambda i: (i, 0)),
        out_shape=jax.ShapeDtypeStruct((n, H), jnp.float32),
    )(h, mx, pp, wx, wd, wp, b)


def _final_body(h_ref, wf_ref, o_ref):
    h = h_ref[...]
    wf = wf_ref[...]
    logits = jnp.sum(h * wf, axis=1, keepdims=True)
    m = jnp.max(logits)
    e = jnp.exp(logits - m)
    o_ref[...] = e / jnp.sum(e)


def _final(h, wf):
    n = h.shape[0]
    return pl.pallas_call(
        _final_body,
        out_shape=jax.ShapeDtypeStruct((n, 1), jnp.float32),
    )(h, wf.reshape(1, H))


def kernel(x, pos, W0, b0, W1, b1, W2, b2, Wf, bf):
    sq = jnp.sum(pos * pos, axis=1)
    posr = pos.astype(jnp.bfloat16).astype(jnp.float32)
    posp = jnp.pad(posr, ((0, NP - N), (0, 0)))
    sqp = jnp.pad(sq, (0, NP - N))
    idx_flat = _knn_sc(posp[:, 0], posp[:, 1], posp[:, 2], sqp)
    idx = idx_flat.reshape(NP, OW)[:N, :K]

    nbr2 = idx[idx]
    mut = jnp.any(nbr2 == jnp.arange(N)[:, None, None], axis=-1)
    w = 1.0 - mut.astype(jnp.float32)
    flat_t = idx.reshape(-1)
    cnt = K + jax.ops.segment_sum(w.reshape(-1), flat_t, num_segments=N)

    def agg(h):
        g = h[idx].sum(1)
        s = jax.ops.segment_sum(
            jnp.repeat(h, K, axis=0) * w.reshape(-1)[:, None],
            flat_t, num_segments=N)
        return (g + s) / cnt[:, None]

    m_p = agg(pos)
    pp = jnp.pad(pos - m_p, ((0, 0), (0, 5)))

    h = x
    for W, b in [(W0, b0), (W1, b1), (W2, b2)]:
        d = h.shape[1]
        wx, wd = W[:d], W[d:2 * d]
        wp = jnp.pad(W[2 * d:], ((0, 5), (0, 0)))
        m_x = agg(h)
        h = _layer(h, m_x, pp, wx, wd, wp, b.reshape(1, H))

    return _final(h, Wf).reshape(N)


# speedup vs baseline: 2.5479x; 1.1270x over previous
import functools

import jax
import jax.numpy as jnp
from jax import lax
from jax.experimental import pallas as pl
from jax.experimental.pallas import tpu as pltpu
from jax.experimental.pallas import tpu_sc as plsc

N = 10000
K = 20
H = 128

NP = 10016
RPW = NP // 32
OW = 32
NV = N // 16
CAND = 512
BIG = 3.0e38


def _knn_body(px_h, py_h, pz_h, sq_h, out_h, px, py, pz, sq, drow,
              cd, ci, ob):
    wid = lax.axis_index("s") * 2 + lax.axis_index("c")
    pltpu.sync_copy(px_h, px)
    pltpu.sync_copy(py_h, py)
    pltpu.sync_copy(pz_h, pz)
    pltpu.sync_copy(sq_h, sq)

    iota = lax.iota(jnp.int32, 16)
    infv = jnp.full((16,), BIG, jnp.float32)
    negv = jnp.full((16,), -1, jnp.int32)
    bigp = jnp.full((16,), 1 << 30, jnp.int32)
    base = wid * RPW

    def shuf(v, lanes):
        return lax.gather(
            v, lanes[:, None],
            lax.GatherDimensionNumbers(offset_dims=(), collapsed_slice_dims=(0,),
                                       start_index_map=(0,)),
            (1,), mode=lax.GatherScatterMode.PROMISE_IN_BOUNDS)

    def splat_at(ref, p):
        blk = (p // 16) * 16
        v = ref[pl.ds(blk, 16)]
        return shuf(v, jnp.full((16,), p - blk, jnp.int32))

    def row_body(rl, _):
        r = base + rl
        s = splat_at(sq, r)
        a = -2.0 * splat_at(px, r)
        b = -2.0 * splat_at(py, r)
        c = -2.0 * splat_at(pz, r)

        def dpair(j):
            jb = 16 * j
            xv = px[pl.ds(jb, 16)]
            yv = py[pl.ds(jb, 16)]
            zv = pz[pl.ds(jb, 16)]
            tv = sq[pl.ds(jb, 16)]
            d = (tv + s) + ((a * xv + b * yv) + c * zv)
            jv = iota + jb
            d = jnp.where(jv == r, BIG, d)
            drow[pl.ds(jb, 16)] = d
            return d

        def dbody(i, carry):
            m1, m2 = carry
            m1 = jnp.minimum(m1, dpair(2 * i))
            m2 = jnp.minimum(m2, dpair(2 * i + 1))
            return m1, m2

        m1, m2 = lax.fori_loop(0, NV // 2, dbody, (infv, infv), unroll=4)
        m1 = jnp.minimum(m1, dpair(NV - 1))

        s1 = jnp.sort(m1)
        s2 = jnp.sort(m2)
        tau = jnp.maximum(s1[9], s2[9])

        zerov = jnp.zeros((16,), jnp.int32)

        def cbody(v, off):
            jb = 16 * v
            d = drow[pl.ds(jb, 16)]
            mask = d <= tau
            m01 = jnp.where(mask, 1, 0)
            csum = plsc.cumsum(m01)
            pos = jnp.minimum(off + csum - 1, CAND - 1)
            plsc.store_scatter(cd, [pos], d, mask=mask)
            plsc.store_scatter(ci, [pos], iota + jb, mask=mask)
            tot = m01
            for sh in (8, 4, 2, 1):
                tot = tot + shuf(tot, iota ^ sh)
            return off + tot

        off_v = lax.fori_loop(0, NV, cbody, zerov, unroll=4)
        nc = jnp.minimum(off_v[0], CAND)
        cd[pl.ds(nc, 16)] = infv
        nv = (nc + 15) // 16

        def fold2(v, carry):
            fv, fp = carry
            d = cd[pl.ds(16 * v, 16)]
            lt = d < fv
            return jnp.where(lt, d, fv), jnp.where(lt, 16 * v + iota, fp)

        acc1 = negv
        acc2 = negv
        for k in range(K):
            fv, fp = lax.fori_loop(0, nv, fold2, (infv, bigp))
            for sh in (8, 4, 2, 1):
                ov = shuf(fv, iota ^ sh)
                op = shuf(fp, iota ^ sh)
                sw = (ov < fv) | ((ov == fv) & (op < fp))
                fv = jnp.where(sw, ov, fv)
                fp = jnp.where(sw, op, fp)
            p = fp[0]
            idxs = splat_at(ci, p)
            if k < 16:
                acc1 = jnp.where(iota == k, idxs, acc1)
            else:
                acc2 = jnp.where(iota == (k - 16), idxs, acc2)
            blk = (p // 16) * 16
            vv = cd[pl.ds(blk, 16)]
            cd[pl.ds(blk, 16)] = jnp.where(iota == p - blk, BIG, vv)
        ob[pl.ds(rl * OW, 16)] = acc1
        ob[pl.ds(rl * OW + 16, 16)] = acc2
        return 0

    lax.fori_loop(0, RPW, row_body, 0)
    pltpu.sync_copy(ob, out_h.at[pl.ds(base * OW, RPW * OW)])


@jax.jit
def _knn_sc(px, py, pz, sq):
    mesh = plsc.VectorSubcoreMesh(core_axis_name="c", subcore_axis_name="s")
    fn = pl.kernel(
        _knn_body,
        mesh=mesh,
        compiler_params=pltpu.CompilerParams(needs_layout_passes=False),
        out_type=jax.ShapeDtypeStruct((NP * OW,), jnp.int32),
        scratch_types=[
            pltpu.VMEM((NP,), jnp.float32),
            pltpu.VMEM((NP,), jnp.float32),
            pltpu.VMEM((NP,), jnp.float32),
            pltpu.VMEM((NP,), jnp.float32),
            pltpu.VMEM((N,), jnp.float32),
            pltpu.VMEM((CAND + 16,), jnp.float32),
            pltpu.VMEM((CAND,), jnp.int32),
            pltpu.VMEM((RPW * OW,), jnp.int32),
        ],
    )
    return fn(px, py, pz, sq)


def _layer_body(h_ref, mx_ref, pp_ref, wx_ref, wd_ref, wp_ref, b_ref, o_ref):
    h = h_ref[...]
    mx = mx_ref[...]
    acc = jnp.dot(mx, wx_ref[...], preferred_element_type=jnp.float32,
                  precision=lax.Precision.HIGHEST)
    acc += jnp.dot(h - mx, wd_ref[...], preferred_element_type=jnp.float32,
                   precision=lax.Precision.HIGHEST)
    acc += jnp.dot(pp_ref[...], wp_ref[...], preferred_element_type=jnp.float32,
                   precision=lax.Precision.HIGHEST)
    o_ref[...] = jnp.maximum(acc + b_ref[...], 0.0)


@functools.partial(jax.jit, static_argnames=("blk",))
def _layer(h, mx, pp, wx, wd, wp, b, blk=1000):
    n = h.shape[0]
    grid = (n // blk,)
    return pl.pallas_call(
        _layer_body,
        grid=grid,
        in_specs=[
            pl.BlockSpec((blk, H), lambda i: (i, 0)),
            pl.BlockSpec((blk, H), lambda i: (i, 0)),
            pl.BlockSpec((blk, 8), lambda i: (i, 0)),
            pl.BlockSpec((H, H), lambda i: (0, 0)),
            pl.BlockSpec((H, H), lambda i: (0, 0)),
            pl.BlockSpec((8, H), lambda i: (0, 0)),
            pl.BlockSpec((1, H), lambda i: (0, 0)),
        ],
        out_specs=pl.BlockSpec((blk, H), lambda i: (i, 0)),
        out_shape=jax.ShapeDtypeStruct((n, H), jnp.float32),
    )(h, mx, pp, wx, wd, wp, b)


def _final_body(h_ref, wf_ref, o_ref):
    h = h_ref[...]
    wf = wf_ref[...]
    logits = jnp.sum(h * wf, axis=1, keepdims=True)
    m = jnp.max(logits)
    e = jnp.exp(logits - m)
    o_ref[...] = e / jnp.sum(e)


def _final(h, wf):
    n = h.shape[0]
    return pl.pallas_call(
        _final_body,
        out_shape=jax.ShapeDtypeStruct((n, 1), jnp.float32),
    )(h, wf.reshape(1, H))


def kernel(x, pos, W0, b0, W1, b1, W2, b2, Wf, bf):
    sq = jnp.sum(pos * pos, axis=1)
    posr = pos.astype(jnp.bfloat16).astype(jnp.float32)
    posp = jnp.pad(posr, ((0, NP - N), (0, 0)))
    sqp = jnp.pad(sq, (0, NP - N))
    idx_flat = _knn_sc(posp[:, 0], posp[:, 1], posp[:, 2], sqp)
    idx = idx_flat.reshape(NP, OW)[:N, :K]

    nbr2 = idx[idx]
    mut = jnp.any(nbr2 == jnp.arange(N)[:, None, None], axis=-1)
    w = 1.0 - mut.astype(jnp.float32)
    flat_t = idx.reshape(-1)
    cnt = K + jax.ops.segment_sum(w.reshape(-1), flat_t, num_segments=N)

    def agg(h):
        g = h[idx].sum(1)
        s = jax.ops.segment_sum(
            jnp.repeat(h, K, axis=0) * w.reshape(-1)[:, None],
            flat_t, num_segments=N)
        return (g + s) / cnt[:, None]

    m_p = agg(pos)
    pp = jnp.pad(pos - m_p, ((0, 0), (0, 5)))

    h = x
    for W, b in [(W0, b0), (W1, b1), (W2, b2)]:
        d = h.shape[1]
        wx, wd = W[:d], W[d:2 * d]
        wp = jnp.pad(W[2 * d:], ((0, 5), (0, 0)))
        m_x = agg(h)
        h = _layer(h, m_x, pp, wx, wd, wp, b.reshape(1, H))

    return _final(h, Wf).reshape(N)


# speedup vs baseline: 3.0552x; 1.1991x over previous
import functools

import jax
import jax.numpy as jnp
from jax import lax
from jax.experimental import pallas as pl
from jax.experimental.pallas import tpu as pltpu
from jax.experimental.pallas import tpu_sc as plsc

N = 10000
K = 20
H = 128

NP = 10016
RPW = NP // 32
OW = 32
NV = N // 16
CAND = 512
BIG = 3.0e38


def _knn_body(px_h, py_h, pz_h, sq_h, out_h, px, py, pz, sq, drow,
              cd, ci, ob):
    wid = lax.axis_index("s") * 2 + lax.axis_index("c")
    pltpu.sync_copy(px_h, px)
    pltpu.sync_copy(py_h, py)
    pltpu.sync_copy(pz_h, pz)
    pltpu.sync_copy(sq_h, sq)

    iota = lax.iota(jnp.int32, 16)
    infv = jnp.full((16,), BIG, jnp.float32)
    negv = jnp.full((16,), -1, jnp.int32)
    bigp = jnp.full((16,), 1 << 30, jnp.int32)
    base = wid * RPW

    def shuf(v, lanes):
        return lax.gather(
            v, lanes[:, None],
            lax.GatherDimensionNumbers(offset_dims=(), collapsed_slice_dims=(0,),
                                       start_index_map=(0,)),
            (1,), mode=lax.GatherScatterMode.PROMISE_IN_BOUNDS)

    def splat_at(ref, p):
        blk = (p // 16) * 16
        v = ref[pl.ds(blk, 16)]
        return shuf(v, jnp.full((16,), p - blk, jnp.int32))

    def row_body(rl, _):
        r = base + rl
        s = splat_at(sq, r)
        a = -2.0 * splat_at(px, r)
        b = -2.0 * splat_at(py, r)
        c = -2.0 * splat_at(pz, r)

        def dpair(j):
            jb = 16 * j
            xv = px[pl.ds(jb, 16)]
            yv = py[pl.ds(jb, 16)]
            zv = pz[pl.ds(jb, 16)]
            tv = sq[pl.ds(jb, 16)]
            d = (tv + s) + ((a * xv + b * yv) + c * zv)
            jv = iota + jb
            d = jnp.where(jv == r, BIG, d)
            drow[pl.ds(jb, 16)] = d
            return d

        def dbody(i, carry):
            m1, m2 = carry
            m1 = jnp.minimum(m1, dpair(2 * i))
            m2 = jnp.minimum(m2, dpair(2 * i + 1))
            return m1, m2

        m1, m2 = lax.fori_loop(0, NV // 2, dbody, (infv, infv), unroll=4)
        m1 = jnp.minimum(m1, dpair(NV - 1))

        s1 = jnp.sort(m1)
        s2 = jnp.sort(m2)
        tau = jnp.maximum(s1[9], s2[9])

        zerov = jnp.zeros((16,), jnp.int32)
        lane15 = jnp.full((16,), 15, jnp.int32)

        def cgroup(g, off):
            jb = 64 * g
            ds = [drow[pl.ds(jb + 16 * u, 16)] for u in range(4)]
            masks = [d <= tau for d in ds]
            csums = [plsc.cumsum(jnp.where(m, 1, 0)) for m in masks]
            tots = [shuf(cs, lane15) for cs in csums]
            bases = [off, off + tots[0], off + (tots[0] + tots[1]),
                     off + (tots[0] + tots[1]) + tots[2]]
            for u in range(4):
                pos = jnp.minimum(bases[u] + csums[u] - 1, CAND - 1)
                plsc.store_scatter(cd, [pos], ds[u], mask=masks[u])
                plsc.store_scatter(ci, [pos], iota + (jb + 16 * u),
                                   mask=masks[u])
            return off + ((tots[0] + tots[1]) + (tots[2] + tots[3]))

        off_v = lax.fori_loop(0, NV // 4, cgroup, zerov)

        dt = drow[pl.ds(16 * (NV - 1), 16)]
        maskt = dt <= tau
        csumt = plsc.cumsum(jnp.where(maskt, 1, 0))
        post = jnp.minimum(off_v + csumt - 1, CAND - 1)
        plsc.store_scatter(cd, [post], dt, mask=maskt)
        plsc.store_scatter(ci, [post], iota + 16 * (NV - 1), mask=maskt)
        off_v = off_v + shuf(csumt, lane15)
        nc = jnp.minimum(off_v[0], CAND)
        cd[pl.ds(nc, 16)] = infv
        nv = (nc + 15) // 16

        def fold2(v, carry):
            fv, fp = carry
            d = cd[pl.ds(16 * v, 16)]
            lt = d < fv
            return jnp.where(lt, d, fv), jnp.where(lt, 16 * v + iota, fp)

        acc1 = negv
        acc2 = negv
        for k in range(K):
            fv, fp = lax.fori_loop(0, nv, fold2, (infv, bigp))
            for sh in (8, 4, 2, 1):
                ov = shuf(fv, iota ^ sh)
                op = shuf(fp, iota ^ sh)
                sw = (ov < fv) | ((ov == fv) & (op < fp))
                fv = jnp.where(sw, ov, fv)
                fp = jnp.where(sw, op, fp)
            p = fp[0]
            idxs = splat_at(ci, p)
            if k < 16:
                acc1 = jnp.where(iota == k, idxs, acc1)
            else:
                acc2 = jnp.where(iota == (k - 16), idxs, acc2)
            blk = (p // 16) * 16
            vv = cd[pl.ds(blk, 16)]
            cd[pl.ds(blk, 16)] = jnp.where(iota == p - blk, BIG, vv)
        ob[pl.ds(rl * OW, 16)] = acc1
        ob[pl.ds(rl * OW + 16, 16)] = acc2
        return 0

    lax.fori_loop(0, RPW, row_body, 0)
    pltpu.sync_copy(ob, out_h.at[pl.ds(base * OW, RPW * OW)])


@jax.jit
def _knn_sc(px, py, pz, sq):
    mesh = plsc.VectorSubcoreMesh(core_axis_name="c", subcore_axis_name="s")
    fn = pl.kernel(
        _knn_body,
        mesh=mesh,
        compiler_params=pltpu.CompilerParams(needs_layout_passes=False),
        out_type=jax.ShapeDtypeStruct((NP * OW,), jnp.int32),
        scratch_types=[
            pltpu.VMEM((NP,), jnp.float32),
            pltpu.VMEM((NP,), jnp.float32),
            pltpu.VMEM((NP,), jnp.float32),
            pltpu.VMEM((NP,), jnp.float32),
            pltpu.VMEM((N,), jnp.float32),
            pltpu.VMEM((CAND + 16,), jnp.float32),
            pltpu.VMEM((CAND,), jnp.int32),
            pltpu.VMEM((RPW * OW,), jnp.int32),
        ],
    )
    return fn(px, py, pz, sq)


def _layer_body(h_ref, mx_ref, pp_ref, wx_ref, wd_ref, wp_ref, b_ref, o_ref):
    h = h_ref[...]
    mx = mx_ref[...]
    acc = jnp.dot(mx, wx_ref[...], preferred_element_type=jnp.float32,
                  precision=lax.Precision.HIGHEST)
    acc += jnp.dot(h - mx, wd_ref[...], preferred_element_type=jnp.float32,
                   precision=lax.Precision.HIGHEST)
    acc += jnp.dot(pp_ref[...], wp_ref[...], preferred_element_type=jnp.float32,
                   precision=lax.Precision.HIGHEST)
    o_ref[...] = jnp.maximum(acc + b_ref[...], 0.0)


@functools.partial(jax.jit, static_argnames=("blk",))
def _layer(h, mx, pp, wx, wd, wp, b, blk=1000):
    n = h.shape[0]
    grid = (n // blk,)
    return pl.pallas_call(
        _layer_body,
        grid=grid,
        in_specs=[
            pl.BlockSpec((blk, H), lambda i: (i, 0)),
            pl.BlockSpec((blk, H), lambda i: (i, 0)),
            pl.BlockSpec((blk, 8), lambda i: (i, 0)),
            pl.BlockSpec((H, H), lambda i: (0, 0)),
            pl.BlockSpec((H, H), lambda i: (0, 0)),
            pl.BlockSpec((8, H), lambda i: (0, 0)),
            pl.BlockSpec((1, H), lambda i: (0, 0)),
        ],
        out_specs=pl.BlockSpec((blk, H), lambda i: (i, 0)),
        out_shape=jax.ShapeDtypeStruct((n, H), jnp.float32),
    )(h, mx, pp, wx, wd, wp, b)


def _final_body(h_ref, wf_ref, o_ref):
    h = h_ref[...]
    wf = wf_ref[...]
    logits = jnp.sum(h * wf, axis=1, keepdims=True)
    m = jnp.max(logits)
    e = jnp.exp(logits - m)
    o_ref[...] = e / jnp.sum(e)


def _final(h, wf):
    n = h.shape[0]
    return pl.pallas_call(
        _final_body,
        out_shape=jax.ShapeDtypeStruct((n, 1), jnp.float32),
    )(h, wf.reshape(1, H))


def kernel(x, pos, W0, b0, W1, b1, W2, b2, Wf, bf):
    sq = jnp.sum(pos * pos, axis=1)
    posr = pos.astype(jnp.bfloat16).astype(jnp.float32)
    posp = jnp.pad(posr, ((0, NP - N), (0, 0)))
    sqp = jnp.pad(sq, (0, NP - N))
    idx_flat = _knn_sc(posp[:, 0], posp[:, 1], posp[:, 2], sqp)
    idx = idx_flat.reshape(NP, OW)[:N, :K]

    nbr2 = idx[idx]
    mut = jnp.any(nbr2 == jnp.arange(N)[:, None, None], axis=-1)
    w = 1.0 - mut.astype(jnp.float32)
    flat_t = idx.reshape(-1)
    cnt = K + jax.ops.segment_sum(w.reshape(-1), flat_t, num_segments=N)

    def agg(h):
        g = h[idx].sum(1)
        s = jax.ops.segment_sum(
            jnp.repeat(h, K, axis=0) * w.reshape(-1)[:, None],
            flat_t, num_segments=N)
        return (g + s) / cnt[:, None]

    m_p = agg(pos)
    pp = jnp.pad(pos - m_p, ((0, 0), (0, 5)))

    h = x
    for W, b in [(W0, b0), (W1, b1), (W2, b2)]:
        d = h.shape[1]
        wx, wd = W[:d], W[d:2 * d]
        wp = jnp.pad(W[2 * d:], ((0, 5), (0, 0)))
        m_x = agg(h)
        h = _layer(h, m_x, pp, wx, wd, wp, b.reshape(1, H))

    return _final(h, Wf).reshape(N)


# speedup vs baseline: 3.2631x; 1.0680x over previous
import functools

import jax
import jax.numpy as jnp
from jax import lax
from jax.experimental import pallas as pl
from jax.experimental.pallas import tpu as pltpu
from jax.experimental.pallas import tpu_sc as plsc

N = 10000
K = 20
H = 128

NP = 10016
RPW = NP // 32
OW = 32
NV = N // 16
CAND = 512
BIG = 3.0e38


def _knn_body(px_h, py_h, pz_h, sq_h, out_h, px, py, pz, sq, drow,
              cd, ci, ob):
    wid = lax.axis_index("s") * 2 + lax.axis_index("c")
    pltpu.sync_copy(px_h, px)
    pltpu.sync_copy(py_h, py)
    pltpu.sync_copy(pz_h, pz)
    pltpu.sync_copy(sq_h, sq)

    iota = lax.iota(jnp.int32, 16)
    infv = jnp.full((16,), BIG, jnp.float32)
    negv = jnp.full((16,), -1, jnp.int32)
    bigp = jnp.full((16,), 1 << 30, jnp.int32)
    base = wid * RPW

    def shuf(v, lanes):
        return lax.gather(
            v, lanes[:, None],
            lax.GatherDimensionNumbers(offset_dims=(), collapsed_slice_dims=(0,),
                                       start_index_map=(0,)),
            (1,), mode=lax.GatherScatterMode.PROMISE_IN_BOUNDS)

    def splat_at(ref, p):
        blk = (p // 16) * 16
        v = ref[pl.ds(blk, 16)]
        return shuf(v, jnp.full((16,), p - blk, jnp.int32))

    def row_body(rl, _):
        r = base + rl
        s = splat_at(sq, r)
        a = -2.0 * splat_at(px, r)
        b = -2.0 * splat_at(py, r)
        c = -2.0 * splat_at(pz, r)

        def dpair(j):
            jb = 16 * j
            xv = px[pl.ds(jb, 16)]
            yv = py[pl.ds(jb, 16)]
            zv = pz[pl.ds(jb, 16)]
            tv = sq[pl.ds(jb, 16)]
            d = (tv + s) + ((a * xv + b * yv) + c * zv)
            jv = iota + jb
            d = jnp.where(jv == r, BIG, d)
            drow[pl.ds(jb, 16)] = d
            return d

        def dbody(i, carry):
            m1, m2 = carry
            m1 = jnp.minimum(m1, dpair(2 * i))
            m2 = jnp.minimum(m2, dpair(2 * i + 1))
            return m1, m2

        m1, m2 = lax.fori_loop(0, NV // 2, dbody, (infv, infv), unroll=4)
        m1 = jnp.minimum(m1, dpair(NV - 1))

        s1 = jnp.sort(m1)
        s2 = jnp.sort(m2)
        tau = jnp.maximum(s1[9], s2[9])

        zerov = jnp.zeros((16,), jnp.int32)
        lane15 = jnp.full((16,), 15, jnp.int32)

        def cgroup(g, off):
            jb = 64 * g
            ds = [drow[pl.ds(jb + 16 * u, 16)] for u in range(4)]
            masks = [d <= tau for d in ds]
            csums = [plsc.cumsum(jnp.where(m, 1, 0)) for m in masks]
            tots = [shuf(cs, lane15) for cs in csums]
            bases = [off, off + tots[0], off + (tots[0] + tots[1]),
                     off + (tots[0] + tots[1]) + tots[2]]
            for u in range(4):
                pos = jnp.minimum(bases[u] + csums[u] - 1, CAND - 1)
                plsc.store_scatter(cd, [pos], ds[u], mask=masks[u])
                plsc.store_scatter(ci, [pos], iota + (jb + 16 * u),
                                   mask=masks[u])
            return off + ((tots[0] + tots[1]) + (tots[2] + tots[3]))

        off_v = lax.fori_loop(0, NV // 4, cgroup, zerov)

        dt = drow[pl.ds(16 * (NV - 1), 16)]
        maskt = dt <= tau
        csumt = plsc.cumsum(jnp.where(maskt, 1, 0))
        post = jnp.minimum(off_v + csumt - 1, CAND - 1)
        plsc.store_scatter(cd, [post], dt, mask=maskt)
        plsc.store_scatter(ci, [post], iota + 16 * (NV - 1), mask=maskt)
        off_v = off_v + shuf(csumt, lane15)
        nc = jnp.minimum(off_v[0], CAND)
        cd[pl.ds(nc, 16)] = infv
        nv = (nc + 15) // 16

        def fold2(v, carry):
            fv, fp = carry
            d = cd[pl.ds(16 * v, 16)]
            lt = d < fv
            return jnp.where(lt, d, fv), jnp.where(lt, 16 * v + iota, fp)

        acc1 = negv
        acc2 = negv
        for k in range(K):
            fv, fp = lax.fori_loop(0, nv, fold2, (infv, bigp))
            for sh in (8, 4, 2, 1):
                ov = shuf(fv, iota ^ sh)
                op = shuf(fp, iota ^ sh)
                sw = (ov < fv) | ((ov == fv) & (op < fp))
                fv = jnp.where(sw, ov, fv)
                fp = jnp.where(sw, op, fp)
            p = fp[0]
            idxs = splat_at(ci, p)
            if k < 16:
                acc1 = jnp.where(iota == k, idxs, acc1)
            else:
                acc2 = jnp.where(iota == (k - 16), idxs, acc2)
            blk = (p // 16) * 16
            vv = cd[pl.ds(blk, 16)]
            cd[pl.ds(blk, 16)] = jnp.where(iota == p - blk, BIG, vv)
        ob[pl.ds(rl * OW, 16)] = acc1
        ob[pl.ds(rl * OW + 16, 16)] = acc2
        return 0

    lax.fori_loop(0, RPW, row_body, 0)
    pltpu.sync_copy(ob, out_h.at[pl.ds(base * OW, RPW * OW)])


@jax.jit
def _knn_sc(px, py, pz, sq):
    mesh = plsc.VectorSubcoreMesh(core_axis_name="c", subcore_axis_name="s")
    fn = pl.kernel(
        _knn_body,
        mesh=mesh,
        compiler_params=pltpu.CompilerParams(needs_layout_passes=False),
        out_type=jax.ShapeDtypeStruct((NP * OW,), jnp.int32),
        scratch_types=[
            pltpu.VMEM((NP,), jnp.float32),
            pltpu.VMEM((NP,), jnp.float32),
            pltpu.VMEM((NP,), jnp.float32),
            pltpu.VMEM((NP,), jnp.float32),
            pltpu.VMEM((N,), jnp.float32),
            pltpu.VMEM((CAND + 16,), jnp.float32),
            pltpu.VMEM((CAND,), jnp.int32),
            pltpu.VMEM((RPW * OW,), jnp.int32),
        ],
    )
    return fn(px, py, pz, sq)


def _layer_body(h_ref, mx_ref, pp_ref, wx_ref, wd_ref, wp_ref, b_ref, o_ref):
    h = h_ref[...]
    mx = mx_ref[...]
    acc = jnp.dot(mx, wx_ref[...], preferred_element_type=jnp.float32,
                  precision=lax.Precision.HIGHEST)
    acc += jnp.dot(h - mx, wd_ref[...], preferred_element_type=jnp.float32,
                   precision=lax.Precision.HIGHEST)
    acc += jnp.dot(pp_ref[...], wp_ref[...], preferred_element_type=jnp.float32,
                   precision=lax.Precision.HIGHEST)
    o_ref[...] = jnp.maximum(acc + b_ref[...], 0.0)


@functools.partial(jax.jit, static_argnames=("blk",))
def _layer(h, mx, pp, wx, wd, wp, b, blk=1000):
    n = h.shape[0]
    grid = (n // blk,)
    return pl.pallas_call(
        _layer_body,
        grid=grid,
        in_specs=[
            pl.BlockSpec((blk, H), lambda i: (i, 0)),
            pl.BlockSpec((blk, H), lambda i: (i, 0)),
            pl.BlockSpec((blk, 8), lambda i: (i, 0)),
            pl.BlockSpec((H, H), lambda i: (0, 0)),
            pl.BlockSpec((H, H), lambda i: (0, 0)),
            pl.BlockSpec((8, H), lambda i: (0, 0)),
            pl.BlockSpec((1, H), lambda i: (0, 0)),
        ],
        out_specs=pl.BlockSpec((blk, H), lambda i: (i, 0)),
        out_shape=jax.ShapeDtypeStruct((n, H), jnp.float32),
    )(h, mx, pp, wx, wd, wp, b)


def _final_body(h_ref, wf_ref, o_ref):
    h = h_ref[...]
    wf = wf_ref[...]
    logits = jnp.sum(h * wf, axis=1, keepdims=True)
    m = jnp.max(logits)
    e = jnp.exp(logits - m)
    o_ref[...] = e / jnp.sum(e)


def _final(h, wf):
    n = h.shape[0]
    return pl.pallas_call(
        _final_body,
        out_shape=jax.ShapeDtypeStruct((n, 1), jnp.float32),
    )(h, wf.reshape(1, H))


def kernel(x, pos, W0, b0, W1, b1, W2, b2, Wf, bf):
    sq = jnp.sum(pos * pos, axis=1)
    posr = pos.astype(jnp.bfloat16).astype(jnp.float32)
    posp = jnp.pad(posr, ((0, NP - N), (0, 0)))
    sqp = jnp.pad(sq, (0, NP - N))
    idx_flat = _knn_sc(posp[:, 0], posp[:, 1], posp[:, 2], sqp)
    idx = idx_flat.reshape(NP, OW)[:N, :K]

    nbr2 = idx[idx]
    mut = jnp.any(nbr2 == jnp.arange(N)[:, None, None], axis=-1)
    w = 1.0 - mut.astype(jnp.float32)
    flat_t = idx.reshape(-1)
    cnt = K + jax.ops.segment_sum(w.reshape(-1), flat_t, num_segments=N)

    t_eff = jnp.where(mut, N, idx).reshape(-1)

    def agg(h):
        g = h[idx].sum(1)
        s = jax.ops.segment_sum(jnp.repeat(h, K, axis=0), t_eff, num_segments=N)
        return (g + s) / cnt[:, None]

    m_p = agg(pos)
    pp = jnp.pad(pos - m_p, ((0, 0), (0, 5)))

    h = x
    for W, b in [(W0, b0), (W1, b1), (W2, b2)]:
        d = h.shape[1]
        wx, wd = W[:d], W[d:2 * d]
        wp = jnp.pad(W[2 * d:], ((0, 5), (0, 0)))
        m_x = agg(h)
        h = _layer(h, m_x, pp, wx, wd, wp, b.reshape(1, H))

    return _final(h, Wf).reshape(N)
